# baseline (speedup 1.0000x reference)
import jax
import jax.numpy as jnp
from jax.experimental import pallas as pl
from jax.experimental.pallas import tpu as pltpu

_EMBED = 16
_H = 96
_N = 64
_STEPS = 4
_NEG = -1e9
_CDT = jnp.bfloat16
_INTERPRET = False


def _relu(v):
    return jnp.maximum(v, 0.0)


def _mx(a, b):
    return jax.lax.dot_general(
        a.astype(_CDT), b.astype(_CDT),
        (((1,), (0,)), ((), ())), preferred_element_type=jnp.float32)


def _rrn_kernel(eo_ref, y1h_ref, erc_ref, gcat_ref, gdt_ref,
                exw1i_ref, exw1rc_ref, exb1_ref, exw2_ref, exb2_ref,
                exw3_ref, exb3_ref,
                mm1a_ref, mm1b_ref, mmb1_ref, mm2_ref, mmb2_ref,
                mm3_ref, mmb3_ref,
                li1f_ref, li1x_ref, lib1_ref, li2_ref, lib2_ref,
                li3_ref, lib3_ref,
                wih_ref, whh_ref, bg_ref, outw_ref, outb_ref,
                o_ref, loss_ref):
    eo = eo_ref[0]
    y1h = y1h_ref[0]
    erc = erc_ref[...]
    gcat = gcat_ref[...]
    gdt = gdt_ref[...]

    exb1 = exb1_ref[...]
    exb2 = exb2_ref[...]
    exb3 = exb3_ref[...]
    mmb1 = mmb1_ref[...]
    mmb2 = mmb2_ref[...]
    mmb3 = mmb3_ref[...]
    lib2 = lib2_ref[...]
    lib3 = lib3_ref[...]
    bg = bg_ref[...]
    outb = outb_ref[...]

    x = _relu(_mx(eo, exw1i_ref[...]) + _mx(erc, exw1rc_ref[...]) + exb1)
    x = _relu(_mx(x, exw2_ref[...]) + exb2)
    x = _mx(x, exw3_ref[...]) + exb3

    xli = _mx(x, li1x_ref[...]) + lib1_ref[...]

    mm1a = mm1a_ref[...]
    mm1b = mm1b_ref[...]
    mm2w = mm2_ref[...]
    mm3w = mm3_ref[...]
    li1f = li1f_ref[...]
    li2w = li2_ref[...]
    li3w = li3_ref[...]
    wih = wih_ref[...]
    whh = whh_ref[...]
    outw = outw_ref[...]

    hm = x
    h = None
    c = None
    acc = jnp.zeros((), jnp.float32)
    o16 = None
    for t in range(_STEPS):
        ab = jnp.concatenate([_mx(hm, mm1a), _mx(hm, mm1b)], axis=0)
        m1 = _relu(_mx(gcat, ab) + mmb1)
        m2 = _relu(_mx(m1, mm2w) + mmb2)
        msgs = _mx(m2, mm3w) + mmb3
        fin = _mx(gdt, msgs)
        li1 = _relu(_mx(fin, li1f) + xli)
        li2 = _relu(_mx(li1, li2w) + lib2)
        il = _mx(li2, li3w) + lib3
        gates = _mx(il, wih) + bg
        if t > 0:
            gates = gates + _mx(h, whh)
        ig = gates[:, 0:_H]
        fg = gates[:, _H:2 * _H]
        gg = gates[:, 2 * _H:3 * _H]
        og = gates[:, 3 * _H:4 * _H]
        newc = jax.nn.sigmoid(ig) * jnp.tanh(gg)
        if t > 0:
            newc = newc + jax.nn.sigmoid(fg) * c
        c = newc
        h = jax.nn.sigmoid(og) * jnp.tanh(c)
        hm = h
        o16 = _mx(c, outw) + outb
        mmax = jnp.max(o16, axis=1, keepdims=True)
        lse = jnp.log(jnp.sum(jnp.exp(o16 - mmax), axis=1, keepdims=True)) + mmax
        acc = acc + jnp.sum((o16 - lse) * y1h)

    o_ref[0] = o16
    loss_ref[...] = jnp.broadcast_to(acc, (1, 1, 128)).astype(jnp.float32)


def kernel(inp, y_true, edges, row_col,
           ex_w1, ex_b1, ex_w2, ex_b2, ex_w3, ex_b3,
           mm_w1, mm_b1, mm_w2, mm_b2, mm_w3, mm_b3,
           li_w1, li_b1, li_w2, li_b2, li_w3, li_b3,
           out_w, out_b, lstm_wih, lstm_whh, lstm_bih, lstm_bhh):
    f32 = jnp.float32
    bs = inp.shape[0]
    e = edges.shape[0]
    inp = inp.astype(jnp.int32)
    y2 = y_true.astype(jnp.int32).reshape(bs, _N)

    eo = jax.nn.one_hot(inp, _EMBED, dtype=f32)
    y1h = jax.nn.one_hot(y2, _EMBED, dtype=f32)
    erc = jnp.concatenate(
        [jax.nn.one_hot(row_col[:, 0], _EMBED, dtype=f32),
         jax.nn.one_hot(row_col[:, 1], _EMBED, dtype=f32)], axis=1)
    gsrc = jax.nn.one_hot(edges[:, 0], _N, dtype=f32)
    gdst = jax.nn.one_hot(edges[:, 1], _N, dtype=f32)
    gcat = jnp.concatenate([gsrc, gdst], axis=1).astype(_CDT)
    gdt = gdst.T.astype(_CDT)

    cd = _CDT
    exw1i = ex_w1[:, :_EMBED].T.astype(cd)
    exw1rc = ex_w1[:, _EMBED:].T.astype(cd)
    exw2 = ex_w2.T.astype(cd)
    exw3 = ex_w3.T.astype(cd)
    mm1a = mm_w1[:, :_H].T.astype(cd)
    mm1b = mm_w1[:, _H:].T.astype(cd)
    mm2 = mm_w2.T.astype(cd)
    mm3 = mm_w3.T.astype(cd)
    li1f = li_w1[:, :_H].T.astype(cd)
    li1x = li_w1[:, _H:].T.astype(cd)
    li2 = li_w2.T.astype(cd)
    li3 = li_w3.T.astype(cd)
    wih = lstm_wih.T.astype(cd)
    whh = lstm_whh.T.astype(cd)
    bg = (lstm_bih + lstm_bhh).reshape(1, 4 * _H)
    outw = jnp.zeros((_H, _EMBED), f32).at[:, :9].set(out_w.T).astype(cd)
    outb = jnp.full((1, _EMBED), _NEG, f32).at[0, :9].set(out_b)

    b1 = ex_b1.reshape(1, _H)
    b2 = ex_b2.reshape(1, _H)
    b3 = ex_b3.reshape(1, _H)
    mb1 = mm_b1.reshape(1, _H)
    mb2 = mm_b2.reshape(1, _H)
    mb3 = mm_b3.reshape(1, _H)
    lb1 = li_b1.reshape(1, _H)
    lb2 = li_b2.reshape(1, _H)
    lb3 = li_b3.reshape(1, _H)

    full = lambda shape: pl.BlockSpec(shape, lambda i: (0,) * len(shape))
    per_b = pl.BlockSpec((1, _N, _EMBED), lambda i: (i, 0, 0))

    o_out, loss_out = pl.pallas_call(
        _rrn_kernel,
        grid=(bs,),
        in_specs=[
            per_b, per_b, full((_N, 32)), full((e, 128)), full((_N, e)),
            full((_EMBED, _H)), full((32, _H)), full((1, _H)),
            full((_H, _H)), full((1, _H)), full((_H, _H)), full((1, _H)),
            full((_H, _H)), full((_H, _H)), full((1, _H)),
            full((_H, _H)), full((1, _H)), full((_H, _H)), full((1, _H)),
            full((_H, _H)), full((_H, _H)), full((1, _H)),
            full((_H, _H)), full((1, _H)), full((_H, _H)), full((1, _H)),
            full((_H, 4 * _H)), full((_H, 4 * _H)), full((1, 4 * _H)),
            full((_H, _EMBED)), full((1, _EMBED)),
        ],
        out_specs=[
            pl.BlockSpec((1, _N, _EMBED), lambda i: (i, 0, 0)),
            pl.BlockSpec((1, 1, 128), lambda i: (i, 0, 0)),
        ],
        out_shape=[
            jax.ShapeDtypeStruct((bs, _N, _EMBED), f32),
            jax.ShapeDtypeStruct((bs, 1, 128), f32),
        ],
        compiler_params=pltpu.CompilerParams(
            dimension_semantics=("parallel",)),
        interpret=_INTERPRET,
    )(eo, y1h, erc, gcat, gdt,
      exw1i, exw1rc, b1, exw2, b2, exw3, b3,
      mm1a, mm1b, mb1, mm2, mb2, mm3, mb3,
      li1f, li1x, lb1, li2, lb2, li3, lb3,
      wih, whh, bg, outw, outb)

    o = o_out.reshape(bs * _N, _EMBED)[:, :9]
    l = -jnp.sum(loss_out[:, 0, 0]) / (bs * _N)
    return (o, l)


# speedup vs baseline: 1.0255x; 1.0255x over previous
import jax
import jax.numpy as jnp
from jax.experimental import pallas as pl
from jax.experimental.pallas import tpu as pltpu

_EMBED = 16
_H = 96
_N = 64
_STEPS = 4
_NEG = -1e9
_CDT = jnp.float32
_BB = 4
_INTERPRET = False


def _relu(v):
    return jnp.maximum(v, 0.0)


def _mx(a, b):
    return jax.lax.dot_general(
        a.astype(_CDT), b.astype(_CDT),
        (((1,), (0,)), ((), ())), preferred_element_type=jnp.float32)


def _rrn_kernel(eo_ref, y1h_ref, erc_ref, gcat_ref, gdt_ref,
                exw1i_ref, exw1rc_ref, exb1_ref, exw2_ref, exb2_ref,
                exw3_ref, exb3_ref,
                mm1a_ref, mm1b_ref, mmb1_ref, mm2_ref, mmb2_ref,
                mm3_ref, mmb3_ref,
                li1f_ref, li1x_ref, lib1_ref, li2_ref, lib2_ref,
                li3_ref, lib3_ref,
                wih_ref, whh_ref, bg_ref, outw_ref, outb_ref,
                o_ref, loss_ref):
    erc = erc_ref[...]
    gcat = gcat_ref[...]
    gdt = gdt_ref[...]

    exb1 = exb1_ref[...]
    exb2 = exb2_ref[...]
    exb3 = exb3_ref[...]
    mmb1 = mmb1_ref[...]
    mmb2 = mmb2_ref[...]
    mmb3 = mmb3_ref[...]
    lib2 = lib2_ref[...]
    lib3 = lib3_ref[...]
    bg = bg_ref[...]
    outb = outb_ref[...]

    exw1i = exw1i_ref[...]
    exw1rc = exw1rc_ref[...]
    exw2 = exw2_ref[...]
    exw3 = exw3_ref[...]
    li1x = li1x_ref[...]
    lib1 = lib1_ref[...]
    mm1a = mm1a_ref[...]
    mm1b = mm1b_ref[...]
    mm2w = mm2_ref[...]
    mm3w = mm3_ref[...]
    li1f = li1f_ref[...]
    li2w = li2_ref[...]
    li3w = li3_ref[...]
    wih = wih_ref[...]
    whh = whh_ref[...]
    outw = outw_ref[...]

    def one_elem(eo, y1h):
        x = _relu(_mx(eo, exw1i) + _mx(erc, exw1rc) + exb1)
        x = _relu(_mx(x, exw2) + exb2)
        x = _mx(x, exw3) + exb3
        xli = _mx(x, li1x) + lib1
        hm = x
        h = None
        c = None
        acc = jnp.zeros((), jnp.float32)
        o16 = None
        for t in range(_STEPS):
            ab = jnp.concatenate([_mx(hm, mm1a), _mx(hm, mm1b)], axis=0)
            m1 = _relu(_mx(gcat, ab) + mmb1)
            m2 = _relu(_mx(m1, mm2w) + mmb2)
            msgs = _mx(m2, mm3w) + mmb3
            fin = _mx(gdt, msgs)
            li1 = _relu(_mx(fin, li1f) + xli)
            li2 = _relu(_mx(li1, li2w) + lib2)
            il = _mx(li2, li3w) + lib3
            gates = _mx(il, wih) + bg
            if t > 0:
                gates = gates + _mx(h, whh)
            ig = gates[:, 0:_H]
            fg = gates[:, _H:2 * _H]
            gg = gates[:, 2 * _H:3 * _H]
            og = gates[:, 3 * _H:4 * _H]
            newc = jax.nn.sigmoid(ig) * jnp.tanh(gg)
            if t > 0:
                newc = newc + jax.nn.sigmoid(fg) * c
            c = newc
            h = jax.nn.sigmoid(og) * jnp.tanh(c)
            hm = h
            o16 = _mx(c, outw) + outb
            mmax = jnp.max(o16, axis=1, keepdims=True)
            lse = jnp.log(
                jnp.sum(jnp.exp(o16 - mmax), axis=1, keepdims=True)) + mmax
            acc = acc + jnp.sum((o16 - lse) * y1h)
        return o16, acc

    tot = jnp.zeros((), jnp.float32)
    for j in range(_BB):
        o16, acc = one_elem(eo_ref[j], y1h_ref[j])
        o_ref[j] = o16
        tot = tot + acc
    loss_ref[...] = jnp.broadcast_to(tot, (1, 1, 128)).astype(jnp.float32)


def kernel(inp, y_true, edges, row_col,
           ex_w1, ex_b1, ex_w2, ex_b2, ex_w3, ex_b3,
           mm_w1, mm_b1, mm_w2, mm_b2, mm_w3, mm_b3,
           li_w1, li_b1, li_w2, li_b2, li_w3, li_b3,
           out_w, out_b, lstm_wih, lstm_whh, lstm_bih, lstm_bhh):
    f32 = jnp.float32
    bs = inp.shape[0]
    e = edges.shape[0]
    inp = inp.astype(jnp.int32)
    y2 = y_true.astype(jnp.int32).reshape(bs, _N)

    eo = jax.nn.one_hot(inp, _EMBED, dtype=f32)
    y1h = jax.nn.one_hot(y2, _EMBED, dtype=f32)
    erc = jnp.concatenate(
        [jax.nn.one_hot(row_col[:, 0], _EMBED, dtype=f32),
         jax.nn.one_hot(row_col[:, 1], _EMBED, dtype=f32)], axis=1)
    gsrc = jax.nn.one_hot(edges[:, 0], _N, dtype=f32)
    gdst = jax.nn.one_hot(edges[:, 1], _N, dtype=f32)
    gcat = jnp.concatenate([gsrc, gdst], axis=1).astype(_CDT)
    gdt = gdst.T.astype(_CDT)

    cd = _CDT
    exw1i = ex_w1[:, :_EMBED].T.astype(cd)
    exw1rc = ex_w1[:, _EMBED:].T.astype(cd)
    exw2 = ex_w2.T.astype(cd)
    exw3 = ex_w3.T.astype(cd)
    mm1a = mm_w1[:, :_H].T.astype(cd)
    mm1b = mm_w1[:, _H:].T.astype(cd)
    mm2 = mm_w2.T.astype(cd)
    mm3 = mm_w3.T.astype(cd)
    li1f = li_w1[:, :_H].T.astype(cd)
    li1x = li_w1[:, _H:].T.astype(cd)
    li2 = li_w2.T.astype(cd)
    li3 = li_w3.T.astype(cd)
    wih = lstm_wih.T.astype(cd)
    whh = lstm_whh.T.astype(cd)
    bg = (lstm_bih + lstm_bhh).reshape(1, 4 * _H)
    outw = jnp.zeros((_H, _EMBED), f32).at[:, :9].set(out_w.T).astype(cd)
    outb = jnp.full((1, _EMBED), _NEG, f32).at[0, :9].set(out_b)

    b1 = ex_b1.reshape(1, _H)
    b2 = ex_b2.reshape(1, _H)
    b3 = ex_b3.reshape(1, _H)
    mb1 = mm_b1.reshape(1, _H)
    mb2 = mm_b2.reshape(1, _H)
    mb3 = mm_b3.reshape(1, _H)
    lb1 = li_b1.reshape(1, _H)
    lb2 = li_b2.reshape(1, _H)
    lb3 = li_b3.reshape(1, _H)

    full = lambda shape: pl.BlockSpec(shape, lambda i: (0,) * len(shape))
    per_b = pl.BlockSpec((_BB, _N, _EMBED), lambda i: (i, 0, 0))

    o_out, loss_out = pl.pallas_call(
        _rrn_kernel,
        grid=(bs // _BB,),
        in_specs=[
            per_b, per_b, full((_N, 32)), full((e, 128)), full((_N, e)),
            full((_EMBED, _H)), full((32, _H)), full((1, _H)),
            full((_H, _H)), full((1, _H)), full((_H, _H)), full((1, _H)),
            full((_H, _H)), full((_H, _H)), full((1, _H)),
            full((_H, _H)), full((1, _H)), full((_H, _H)), full((1, _H)),
            full((_H, _H)), full((_H, _H)), full((1, _H)),
            full((_H, _H)), full((1, _H)), full((_H, _H)), full((1, _H)),
            full((_H, 4 * _H)), full((_H, 4 * _H)), full((1, 4 * _H)),
            full((_H, _EMBED)), full((1, _EMBED)),
        ],
        out_specs=[
            pl.BlockSpec((_BB, _N, _EMBED), lambda i: (i, 0, 0)),
            pl.BlockSpec((1, 1, 128), lambda i: (i, 0, 0)),
        ],
        out_shape=[
            jax.ShapeDtypeStruct((bs, _N, _EMBED), f32),
            jax.ShapeDtypeStruct((bs // _BB, 1, 128), f32),
        ],
        compiler_params=pltpu.CompilerParams(
            dimension_semantics=("parallel",)),
        interpret=_INTERPRET,
    )(eo, y1h, erc, gcat, gdt,
      exw1i, exw1rc, b1, exw2, b2, exw3, b3,
      mm1a, mm1b, mb1, mm2, mb2, mm3, mb3,
      li1f, li1x, lb1, li2, lb2, li3, lb3,
      wih, whh, bg, outw, outb)

    o = o_out.reshape(bs * _N, _EMBED)[:, :9]
    l = -jnp.sum(loss_out[:, 0, 0]) / (bs * _N)
    return (o, l)


# speedup vs baseline: 2.7528x; 2.6845x over previous
import jax
import jax.numpy as jnp
from jax.experimental import pallas as pl
from jax.experimental.pallas import tpu as pltpu

_EMBED = 16
_H = 96
_N = 64
_STEPS = 4
_NEG = -1e9
_CDT = jnp.float32
_BB = 4
_INTERPRET = False


def _relu(v):
    return jnp.maximum(v, 0.0)


def _mx(a, b):
    return jax.lax.dot_general(
        a.astype(_CDT), b.astype(_CDT),
        (((1,), (0,)), ((), ())), preferred_element_type=jnp.float32)


def _rrn_kernel(eo_ref, y1h_ref, erc_ref, gcat_ref, gdt_ref,
                exw1i_ref, exw1rc_ref, exb1_ref, exw2_ref, exb2_ref,
                exw3_ref, exb3_ref,
                mm1a_ref, mm1b_ref, mmb1_ref, mm2_ref, mmb2_ref,
                mm3_ref, mmb3_ref,
                li1f_ref, li1x_ref, lib1_ref, li2_ref, lib2_ref,
                li3_ref, lib3_ref,
                wih_ref, whh_ref, bg_ref, outw_ref, outb_ref,
                o_ref, loss_ref):
    erc = erc_ref[...]
    gcat = gcat_ref[...]
    gdt = gdt_ref[...]

    exb1 = exb1_ref[...]
    exb2 = exb2_ref[...]
    exb3 = exb3_ref[...]
    mmb1 = mmb1_ref[...]
    mmb2 = mmb2_ref[...]
    mmb3 = mmb3_ref[...]
    lib2 = lib2_ref[...]
    lib3 = lib3_ref[...]
    bg = bg_ref[...]
    outb = outb_ref[...]

    exw1i = exw1i_ref[...]
    exw1rc = exw1rc_ref[...]
    exw2 = exw2_ref[...]
    exw3 = exw3_ref[...]
    li1x = li1x_ref[...]
    lib1 = lib1_ref[...]
    mm1a = mm1a_ref[...]
    mm1b = mm1b_ref[...]
    mm2w = mm2_ref[...]
    mm3w = mm3_ref[...]
    li1f = li1f_ref[...]
    li2w = li2_ref[...]
    li3w = li3_ref[...]
    wih = wih_ref[...]
    whh = whh_ref[...]
    outw = outw_ref[...]

    B = range(_BB)
    x = [_relu(_mx(eo_ref[j], exw1i) + _mx(erc, exw1rc) + exb1) for j in B]
    x = [_relu(_mx(x[j], exw2) + exb2) for j in B]
    x = [_mx(x[j], exw3) + exb3 for j in B]
    xli = [_mx(x[j], li1x) + lib1 for j in B]
    hm = x
    h = [None] * _BB
    c = [None] * _BB
    acc = jnp.zeros((), jnp.float32)
    o16 = [None] * _BB
    for t in range(_STEPS):
        ab = [jnp.concatenate([_mx(hm[j], mm1a), _mx(hm[j], mm1b)], axis=0)
              for j in B]
        m1 = [_relu(_mx(gcat, ab[j]) + mmb1) for j in B]
        m2 = [_relu(_mx(m1[j], mm2w) + mmb2) for j in B]
        msgs = [_mx(m2[j], mm3w) + mmb3 for j in B]
        fin = [_mx(gdt, msgs[j]) for j in B]
        li1 = [_relu(_mx(fin[j], li1f) + xli[j]) for j in B]
        li2 = [_relu(_mx(li1[j], li2w) + lib2) for j in B]
        il = [_mx(li2[j], li3w) + lib3 for j in B]
        gates = [_mx(il[j], wih) + bg for j in B]
        if t > 0:
            gates = [gates[j] + _mx(h[j], whh) for j in B]
        for j in B:
            g = gates[j]
            ig = g[:, 0:_H]
            fg = g[:, _H:2 * _H]
            gg = g[:, 2 * _H:3 * _H]
            og = g[:, 3 * _H:4 * _H]
            newc = jax.nn.sigmoid(ig) * jnp.tanh(gg)
            if t > 0:
                newc = newc + jax.nn.sigmoid(fg) * c[j]
            c[j] = newc
            h[j] = jax.nn.sigmoid(og) * jnp.tanh(newc)
        hm = h
        o16 = [_mx(c[j], outw) + outb for j in B]
        for j in B:
            mmax = jnp.max(o16[j], axis=1, keepdims=True)
            lse = jnp.log(
                jnp.sum(jnp.exp(o16[j] - mmax), axis=1, keepdims=True)) + mmax
            acc = acc + jnp.sum((o16[j] - lse) * y1h_ref[j])
    for j in B:
        o_ref[j] = o16[j]
    loss_ref[...] = jnp.broadcast_to(acc, (1, 1, 128)).astype(jnp.float32)


def kernel(inp, y_true, edges, row_col,
           ex_w1, ex_b1, ex_w2, ex_b2, ex_w3, ex_b3,
           mm_w1, mm_b1, mm_w2, mm_b2, mm_w3, mm_b3,
           li_w1, li_b1, li_w2, li_b2, li_w3, li_b3,
           out_w, out_b, lstm_wih, lstm_whh, lstm_bih, lstm_bhh):
    f32 = jnp.float32
    bs = inp.shape[0]
    e = edges.shape[0]
    inp = inp.astype(jnp.int32)
    y2 = y_true.astype(jnp.int32).reshape(bs, _N)

    eo = jax.nn.one_hot(inp, _EMBED, dtype=f32)
    y1h = jax.nn.one_hot(y2, _EMBED, dtype=f32)
    erc = jnp.concatenate(
        [jax.nn.one_hot(row_col[:, 0], _EMBED, dtype=f32),
         jax.nn.one_hot(row_col[:, 1], _EMBED, dtype=f32)], axis=1)
    gsrc = jax.nn.one_hot(edges[:, 0], _N, dtype=f32)
    gdst = jax.nn.one_hot(edges[:, 1], _N, dtype=f32)
    gcat = jnp.concatenate([gsrc, gdst], axis=1).astype(_CDT)
    gdt = gdst.T.astype(_CDT)

    cd = _CDT
    exw1i = ex_w1[:, :_EMBED].T.astype(cd)
    exw1rc = ex_w1[:, _EMBED:].T.astype(cd)
    exw2 = ex_w2.T.astype(cd)
    exw3 = ex_w3.T.astype(cd)
    mm1a = mm_w1[:, :_H].T.astype(cd)
    mm1b = mm_w1[:, _H:].T.astype(cd)
    mm2 = mm_w2.T.astype(cd)
    mm3 = mm_w3.T.astype(cd)
    li1f = li_w1[:, :_H].T.astype(cd)
    li1x = li_w1[:, _H:].T.astype(cd)
    li2 = li_w2.T.astype(cd)
    li3 = li_w3.T.astype(cd)
    wih = lstm_wih.T.astype(cd)
    whh = lstm_whh.T.astype(cd)
    bg = (lstm_bih + lstm_bhh).reshape(1, 4 * _H)
    outw = jnp.zeros((_H, _EMBED), f32).at[:, :9].set(out_w.T).astype(cd)
    outb = jnp.full((1, _EMBED), _NEG, f32).at[0, :9].set(out_b)

    b1 = ex_b1.reshape(1, _H)
    b2 = ex_b2.reshape(1, _H)
    b3 = ex_b3.reshape(1, _H)
    mb1 = mm_b1.reshape(1, _H)
    mb2 = mm_b2.reshape(1, _H)
    mb3 = mm_b3.reshape(1, _H)
    lb1 = li_b1.reshape(1, _H)
    lb2 = li_b2.reshape(1, _H)
    lb3 = li_b3.reshape(1, _H)

    full = lambda shape: pl.BlockSpec(shape, lambda i: (0,) * len(shape))
    per_b = pl.BlockSpec((_BB, _N, _EMBED), lambda i: (i, 0, 0))

    o_out, loss_out = pl.pallas_call(
        _rrn_kernel,
        grid=(bs // _BB,),
        in_specs=[
            per_b, per_b, full((_N, 32)), full((e, 128)), full((_N, e)),
            full((_EMBED, _H)), full((32, _H)), full((1, _H)),
            full((_H, _H)), full((1, _H)), full((_H, _H)), full((1, _H)),
            full((_H, _H)), full((_H, _H)), full((1, _H)),
            full((_H, _H)), full((1, _H)), full((_H, _H)), full((1, _H)),
            full((_H, _H)), full((_H, _H)), full((1, _H)),
            full((_H, _H)), full((1, _H)), full((_H, _H)), full((1, _H)),
            full((_H, 4 * _H)), full((_H, 4 * _H)), full((1, 4 * _H)),
            full((_H, _EMBED)), full((1, _EMBED)),
        ],
        out_specs=[
            pl.BlockSpec((_BB, _N, _EMBED), lambda i: (i, 0, 0)),
            pl.BlockSpec((1, 1, 128), lambda i: (i, 0, 0)),
        ],
        out_shape=[
            jax.ShapeDtypeStruct((bs, _N, _EMBED), f32),
            jax.ShapeDtypeStruct((bs // _BB, 1, 128), f32),
        ],
        compiler_params=pltpu.CompilerParams(
            dimension_semantics=("parallel",)),
        interpret=_INTERPRET,
    )(eo, y1h, erc, gcat, gdt,
      exw1i, exw1rc, b1, exw2, b2, exw3, b3,
      mm1a, mm1b, mb1, mm2, mb2, mm3, mb3,
      li1f, li1x, lb1, li2, lb2, li3, lb3,
      wih, whh, bg, outw, outb)

    o = o_out.reshape(bs * _N, _EMBED)[:, :9]
    l = -jnp.sum(loss_out[:, 0, 0]) / (bs * _N)
    return (o, l)


# speedup vs baseline: 3.2811x; 1.1919x over previous
import jax
import jax.numpy as jnp
from jax.experimental import pallas as pl
from jax.experimental.pallas import tpu as pltpu

_EMBED = 16
_H = 96
_N = 64
_STEPS = 4
_NEG = -1e9
_CDT = jnp.float32
_BB = 8
_INTERPRET = False


def _relu(v):
    return jnp.maximum(v, 0.0)


_DN = (((1,), (0,)), ((), ()))


def _mx(a, b):
    return jax.lax.dot_general(
        a.astype(_CDT), b.astype(_CDT), _DN,
        preferred_element_type=jnp.float32)


def _mxb(a, b):
    return jax.lax.dot_general(
        a.astype(_CDT), b.astype(_CDT), _DN,
        preferred_element_type=jnp.float32)


def _rrn_kernel(eo_ref, y1h_ref, erc_ref, gcat_ref, gdt_ref,
                exw1i_ref, exw1rc_ref, exw2_ref, exb2_ref,
                exw3_ref, exb3_ref,
                mm1a_ref, mm1b_ref, mmb1_ref, mm2_ref, mmb2_ref,
                mm3_ref,
                li1f_ref, li1x_ref, lib1_ref, li2_ref, lib2_ref,
                li3_ref, lib3_ref,
                wih_ref, whh_ref, bg_ref, outw_ref, outb_ref,
                o_ref, loss_ref):
    erc = erc_ref[...]
    gcat = gcat_ref[...]
    gdt = gdt_ref[...]

    exb2 = exb2_ref[...]
    exb3 = exb3_ref[...]
    mmb1 = mmb1_ref[...]
    mmb2 = mmb2_ref[...]
    lib2 = lib2_ref[...]
    lib3 = lib3_ref[...]
    bg = bg_ref[...]
    outb = outb_ref[...]

    exw1i = exw1i_ref[...]
    exw1rc = exw1rc_ref[...]
    exw2 = exw2_ref[...]
    exw3 = exw3_ref[...]
    li1x = li1x_ref[...]
    lib1 = lib1_ref[...]
    mm1a = mm1a_ref[...]
    mm1b = mm1b_ref[...]
    mm2w = mm2_ref[...]
    mm3w = mm3_ref[...]
    li1f = li1f_ref[...]
    li2w = li2_ref[...]
    li3w = li3_ref[...]
    wih = wih_ref[...]
    whh = whh_ref[...]
    outw = outw_ref[...]

    B = range(_BB)
    x = [_relu(_mx(eo_ref[j], exw1i) + _mx(erc, exw1rc)) for j in B]
    x = [_relu(_mx(x[j], exw2) + exb2) for j in B]
    x = [_mx(x[j], exw3) + exb3 for j in B]
    xli = [_mx(x[j], li1x) + lib1 for j in B]
    hm = x
    h = [None] * _BB
    c = [None] * _BB
    acc = jnp.zeros((), jnp.float32)
    o16 = [None] * _BB
    for t in range(_STEPS):
        ab = [jnp.concatenate([_mx(hm[j], mm1a) + mmb1, _mx(hm[j], mm1b)],
                              axis=0) for j in B]
        m1 = [_relu(_mxb(gcat, ab[j])) for j in B]
        m2 = [_relu(_mxb(m1[j], mm2w) + mmb2) for j in B]
        msgs = [_mxb(m2[j], mm3w) for j in B]
        fin = [_mx(gdt, msgs[j]) for j in B]
        li1 = [_relu(_mx(fin[j], li1f) + xli[j]) for j in B]
        li2 = [_relu(_mx(li1[j], li2w) + lib2) for j in B]
        il = [_mx(li2[j], li3w) + lib3 for j in B]
        gates = [_mx(il[j], wih) + bg for j in B]
        if t > 0:
            gates = [gates[j] + _mx(h[j], whh) for j in B]
        for j in B:
            g = gates[j]
            ig = g[:, 0:_H]
            fg = g[:, _H:2 * _H]
            gg = g[:, 2 * _H:3 * _H]
            og = g[:, 3 * _H:4 * _H]
            newc = jax.nn.sigmoid(ig) * jnp.tanh(gg)
            if t > 0:
                newc = newc + jax.nn.sigmoid(fg) * c[j]
            c[j] = newc
            h[j] = jax.nn.sigmoid(og) * jnp.tanh(newc)
        hm = h
        o16 = [_mx(c[j], outw) + outb for j in B]
        for j in B:
            mmax = jnp.max(o16[j], axis=1, keepdims=True)
            lse = jnp.log(
                jnp.sum(jnp.exp(o16[j] - mmax), axis=1, keepdims=True)) + mmax
            acc = acc + jnp.sum((o16[j] - lse) * y1h_ref[j])
    for j in B:
        o_ref[j] = o16[j]
    loss_ref[...] = jnp.broadcast_to(acc, (1, 1, 128)).astype(jnp.float32)


def kernel(inp, y_true, edges, row_col,
           ex_w1, ex_b1, ex_w2, ex_b2, ex_w3, ex_b3,
           mm_w1, mm_b1, mm_w2, mm_b2, mm_w3, mm_b3,
           li_w1, li_b1, li_w2, li_b2, li_w3, li_b3,
           out_w, out_b, lstm_wih, lstm_whh, lstm_bih, lstm_bhh):
    f32 = jnp.float32
    bs = inp.shape[0]
    e = edges.shape[0]
    inp = inp.astype(jnp.int32)
    y2 = y_true.astype(jnp.int32).reshape(bs, _N)

    eo = jax.nn.one_hot(inp, _EMBED, dtype=f32)
    y1h = jax.nn.one_hot(y2, _EMBED, dtype=f32)
    erc = jnp.concatenate(
        [jax.nn.one_hot(row_col[:, 0], _EMBED, dtype=f32),
         jax.nn.one_hot(row_col[:, 1], _EMBED, dtype=f32)], axis=1)
    gsrc = jax.nn.one_hot(edges[:, 0], _N, dtype=f32)
    gdst = jax.nn.one_hot(edges[:, 1], _N, dtype=f32)
    gcat = jnp.concatenate([gsrc, gdst], axis=1).astype(_CDT)
    gdt = gdst.T.astype(_CDT)

    cd = _CDT
    exw1i = (ex_w1[:, :_EMBED].T + ex_b1[None, :]).astype(cd)
    exw1rc = ex_w1[:, _EMBED:].T.astype(cd)
    exw2 = ex_w2.T.astype(cd)
    exw3 = ex_w3.T.astype(cd)
    mm1a = mm_w1[:, :_H].T.astype(cd)
    mm1b = mm_w1[:, _H:].T.astype(cd)
    mm2 = mm_w2.T.astype(cd)
    mm3 = mm_w3.T.astype(cd)
    li1f = li_w1[:, :_H].T.astype(cd)
    li1x = li_w1[:, _H:].T.astype(cd)
    li2 = li_w2.T.astype(cd)
    li3 = li_w3.T.astype(cd)
    wih = lstm_wih.T.astype(cd)
    whh = lstm_whh.T.astype(cd)
    bg = (lstm_bih + lstm_bhh).reshape(1, 4 * _H)
    outw = jnp.zeros((_H, _EMBED), f32).at[:, :9].set(out_w.T).astype(cd)
    outb = jnp.full((1, _EMBED), _NEG, f32).at[0, :9].set(out_b)

    b2 = ex_b2.reshape(1, _H)
    b3 = ex_b3.reshape(1, _H)
    mb1 = mm_b1.reshape(1, _H)
    mb2 = mm_b2.reshape(1, _H).astype(_CDT)
    indeg = jnp.sum(gdst, axis=0)
    lb1 = li_b1[None, :] + indeg[:, None] * (mm_b3[None, :] @ li_w1[:, :_H].T)
    lb2 = li_b2.reshape(1, _H)
    lb3 = li_b3.reshape(1, _H)

    full = lambda shape: pl.BlockSpec(shape, lambda i: (0,) * len(shape))
    per_b = pl.BlockSpec((_BB, _N, _EMBED), lambda i: (i, 0, 0))

    o_out, loss_out = pl.pallas_call(
        _rrn_kernel,
        grid=(bs // _BB,),
        in_specs=[
            per_b, per_b, full((_N, 32)), full((e, 128)), full((_N, e)),
            full((_EMBED, _H)), full((32, _H)),
            full((_H, _H)), full((1, _H)), full((_H, _H)), full((1, _H)),
            full((_H, _H)), full((_H, _H)), full((1, _H)),
            full((_H, _H)), full((1, _H)), full((_H, _H)),
            full((_H, _H)), full((_H, _H)), full((_N, _H)),
            full((_H, _H)), full((1, _H)), full((_H, _H)), full((1, _H)),
            full((_H, 4 * _H)), full((_H, 4 * _H)), full((1, 4 * _H)),
            full((_H, _EMBED)), full((1, _EMBED)),
        ],
        out_specs=[
            pl.BlockSpec((_BB, _N, _EMBED), lambda i: (i, 0, 0)),
            pl.BlockSpec((1, 1, 128), lambda i: (i, 0, 0)),
        ],
        out_shape=[
            jax.ShapeDtypeStruct((bs, _N, _EMBED), f32),
            jax.ShapeDtypeStruct((bs // _BB, 1, 128), f32),
        ],
        compiler_params=pltpu.CompilerParams(
            dimension_semantics=("parallel",)),
        interpret=_INTERPRET,
    )(eo, y1h, erc, gcat, gdt,
      exw1i, exw1rc, exw2, b2, exw3, b3,
      mm1a, mm1b, mb1, mm2, mb2, mm3,
      li1f, li1x, lb1, li2, lb2, li3, lb3,
      wih, whh, bg, outw, outb)

    o = o_out.reshape(bs * _N, _EMBED)[:, :9]
    l = -jnp.sum(loss_out[:, 0, 0]) / (bs * _N)
    return (o, l)


# speedup vs baseline: 3.3752x; 1.0287x over previous
import jax
import jax.numpy as jnp
from jax.experimental import pallas as pl
from jax.experimental.pallas import tpu as pltpu

_EMBED = 16
_H = 96
_N = 64
_E = 1088
_STEPS = 4
_NEG = -1e9
_CDT = jnp.float32
_BB = 8


def _relu(v):
    return jnp.maximum(v, 0.0)


_DN = (((1,), (0,)), ((), ()))


def _mx(a, b):
    return jax.lax.dot_general(
        a.astype(_CDT), b.astype(_CDT), _DN,
        preferred_element_type=jnp.float32)


def _rrn_kernel(eo_ref, y1h_ref, erc_ref, gcat_ref, gdt_ref,
                exw1i_ref, exw1rc_ref, exw2_ref, exb2_ref,
                exw3_ref, exb3_ref,
                mm1a_ref, mm1b_ref, mmb1_ref, mm2_ref, mmb2_ref,
                mm3_ref,
                li1f_ref, li1x_ref, lib1_ref, li2_ref, lib2_ref,
                li3_ref, lib3_ref,
                wih_ref, whh_ref, bg_ref, outw_ref, outb_ref,
                o_ref, loss_ref, scr_ref, fin_ref):
    R = _BB * _N
    erc = erc_ref[...]
    gcat = gcat_ref[...]
    gdt = gdt_ref[...]

    exb2 = exb2_ref[...]
    exb3 = exb3_ref[...]
    mmb1 = mmb1_ref[...]
    mmb2 = mmb2_ref[...]
    lib2 = lib2_ref[...]
    lib3 = lib3_ref[...]
    bg = bg_ref[...]
    outb = outb_ref[...]

    exw1i = exw1i_ref[...]
    exw1rc = exw1rc_ref[...]
    exw2 = exw2_ref[...]
    exw3 = exw3_ref[...]
    li1x = li1x_ref[...]
    lib1 = lib1_ref[...]
    mm1a = mm1a_ref[...]
    mm1b = mm1b_ref[...]
    mm2w = mm2_ref[...]
    mm3w = mm3_ref[...]
    li1f = li1f_ref[...]
    li2w = li2_ref[...]
    li3w = li3_ref[...]
    wih = wih_ref[...]
    whh = whh_ref[...]
    outw = outw_ref[...]

    eo = eo_ref[...].reshape(R, _EMBED)
    y1h = y1h_ref[...].reshape(R, _EMBED)

    x = _relu(_mx(eo, exw1i) + _mx(erc, exw1rc))
    x = _relu(_mx(x, exw2) + exb2)
    x = _mx(x, exw3) + exb3
    xli = _mx(x, li1x) + lib1

    hm = x
    h = None
    c = None
    acc = jnp.zeros((), jnp.float32)
    B = range(_BB)
    for t in range(_STEPS):
        av = _mx(hm, mm1a) + mmb1
        bv = _mx(hm, mm1b)
        for j in B:
            abj = jnp.concatenate(
                [av[j * _N:(j + 1) * _N], bv[j * _N:(j + 1) * _N]], axis=0)
            scr_ref[pl.ds(j * _E, _E), :] = _mx(gcat, abj)
        m1 = _relu(scr_ref[...])
        m2 = _relu(_mx(m1, mm2w) + mmb2)
        msgs = _mx(m2, mm3w)
        for j in B:
            fin_ref[pl.ds(j * _N, _N), :] = _mx(
                gdt, msgs[j * _E:(j + 1) * _E])
        fin = fin_ref[...]
        li1 = _relu(_mx(fin, li1f) + xli)
        li2 = _relu(_mx(li1, li2w) + lib2)
        il = _mx(li2, li3w) + lib3
        gates = _mx(il, wih) + bg
        if t > 0:
            gates = gates + _mx(h, whh)
        ig = gates[:, 0:_H]
        fg = gates[:, _H:2 * _H]
        gg = gates[:, 2 * _H:3 * _H]
        og = gates[:, 3 * _H:4 * _H]
        newc = jax.nn.sigmoid(ig) * jnp.tanh(gg)
        if t > 0:
            newc = newc + jax.nn.sigmoid(fg) * c
        c = newc
        h = jax.nn.sigmoid(og) * jnp.tanh(newc)
        hm = h
        o16 = _mx(c, outw) + outb
        mmax = jnp.max(o16, axis=1, keepdims=True)
        lse = jnp.log(jnp.sum(jnp.exp(o16 - mmax), axis=1,
                              keepdims=True)) + mmax
        acc = acc + jnp.sum((o16 - lse) * y1h)

    o_ref[...] = o16.reshape(_BB, _N, _EMBED)
    loss_ref[...] = jnp.broadcast_to(acc, (1, 1, 128)).astype(jnp.float32)


def kernel(inp, y_true, edges, row_col,
           ex_w1, ex_b1, ex_w2, ex_b2, ex_w3, ex_b3,
           mm_w1, mm_b1, mm_w2, mm_b2, mm_w3, mm_b3,
           li_w1, li_b1, li_w2, li_b2, li_w3, li_b3,
           out_w, out_b, lstm_wih, lstm_whh, lstm_bih, lstm_bhh):
    f32 = jnp.float32
    bs = inp.shape[0]
    e = edges.shape[0]
    inp = inp.astype(jnp.int32)
    y2 = y_true.astype(jnp.int32).reshape(bs, _N)

    eo = jax.nn.one_hot(inp, _EMBED, dtype=f32)
    y1h = jax.nn.one_hot(y2, _EMBED, dtype=f32)
    erc = jnp.tile(jnp.concatenate(
        [jax.nn.one_hot(row_col[:, 0], _EMBED, dtype=f32),
         jax.nn.one_hot(row_col[:, 1], _EMBED, dtype=f32)], axis=1),
        (_BB, 1))
    gsrc = jax.nn.one_hot(edges[:, 0], _N, dtype=f32)
    gdst = jax.nn.one_hot(edges[:, 1], _N, dtype=f32)
    gcat = jnp.concatenate([gsrc, gdst], axis=1).astype(_CDT)
    gdt = gdst.T.astype(_CDT)

    cd = _CDT
    exw1i = (ex_w1[:, :_EMBED].T + ex_b1[None, :]).astype(cd)
    exw1rc = ex_w1[:, _EMBED:].T.astype(cd)
    exw2 = ex_w2.T.astype(cd)
    exw3 = ex_w3.T.astype(cd)
    mm1a = mm_w1[:, :_H].T.astype(cd)
    mm1b = mm_w1[:, _H:].T.astype(cd)
    mm2 = mm_w2.T.astype(cd)
    mm3 = mm_w3.T.astype(cd)
    li1f = li_w1[:, :_H].T.astype(cd)
    li1x = li_w1[:, _H:].T.astype(cd)
    li2 = li_w2.T.astype(cd)
    li3 = li_w3.T.astype(cd)
    wih = lstm_wih.T.astype(cd)
    whh = lstm_whh.T.astype(cd)
    bg = (lstm_bih + lstm_bhh).reshape(1, 4 * _H)
    outw = jnp.zeros((_H, _EMBED), f32).at[:, :9].set(out_w.T).astype(cd)
    outb = jnp.full((1, _EMBED), _NEG, f32).at[0, :9].set(out_b)

    b2 = ex_b2.reshape(1, _H)
    b3 = ex_b3.reshape(1, _H)
    mb1 = mm_b1.reshape(1, _H)
    mb2 = mm_b2.reshape(1, _H)
    indeg = jnp.sum(gdst, axis=0)
    lb1 = jnp.tile(
        li_b1[None, :] + indeg[:, None] * (mm_b3[None, :] @ li_w1[:, :_H].T),
        (_BB, 1))
    lb2 = li_b2.reshape(1, _H)
    lb3 = li_b3.reshape(1, _H)

    full = lambda shape: pl.BlockSpec(shape, lambda i: (0,) * len(shape))
    per_b = pl.BlockSpec((_BB, _N, _EMBED), lambda i: (i, 0, 0))
    R = _BB * _N

    o_out, loss_out = pl.pallas_call(
        _rrn_kernel,
        grid=(bs // _BB,),
        in_specs=[
            per_b, per_b, full((R, 32)), full((e, 128)), full((_N, e)),
            full((_EMBED, _H)), full((32, _H)),
            full((_H, _H)), full((1, _H)), full((_H, _H)), full((1, _H)),
            full((_H, _H)), full((_H, _H)), full((1, _H)),
            full((_H, _H)), full((1, _H)), full((_H, _H)),
            full((_H, _H)), full((_H, _H)), full((R, _H)),
            full((_H, _H)), full((1, _H)), full((_H, _H)), full((1, _H)),
            full((_H, 4 * _H)), full((_H, 4 * _H)), full((1, 4 * _H)),
            full((_H, _EMBED)), full((1, _EMBED)),
        ],
        out_specs=[
            pl.BlockSpec((_BB, _N, _EMBED), lambda i: (i, 0, 0)),
            pl.BlockSpec((1, 1, 128), lambda i: (i, 0, 0)),
        ],
        out_shape=[
            jax.ShapeDtypeStruct((bs, _N, _EMBED), f32),
            jax.ShapeDtypeStruct((bs // _BB, 1, 128), f32),
        ],
        scratch_shapes=[
            pltpu.VMEM((_BB * _E, _H), f32),
            pltpu.VMEM((R, _H), f32),
        ],
        compiler_params=pltpu.CompilerParams(
            dimension_semantics=("parallel",)),
        interpret=_INTERPRET,
    )(eo, y1h, erc, gcat, gdt,
      exw1i, exw1rc, exw2, b2, exw3, b3,
      mm1a, mm1b, mb1, mm2, mb2, mm3,
      li1f, li1x, lb1, li2, lb2, li3, lb3,
      wih, whh, bg, outw, outb)

    o = o_out.reshape(bs * _N, _EMBED)[:, :9]
    l = -jnp.sum(loss_out[:, 0, 0]) / (bs * _N)
    return (o, l)


_INTERPRET = False


# speedup vs baseline: 3.7715x; 1.1174x over previous
import jax
import jax.numpy as jnp
from jax.experimental import pallas as pl
from jax.experimental.pallas import tpu as pltpu

_EMBED = 16
_H = 96
_N = 64
_E = 1088
_STEPS = 4
_NEG = -1e9
_CDT = jnp.float32
_BB = 16


def _relu(v):
    return jnp.maximum(v, 0.0)


_DN = (((1,), (0,)), ((), ()))


def _mx(a, b):
    return jax.lax.dot_general(
        a.astype(_CDT), b.astype(_CDT), _DN,
        preferred_element_type=jnp.float32)


def _rrn_kernel(eo_ref, y1h_ref, erc_ref, gcat_ref, gdt_ref,
                exw1i_ref, exw1rc_ref, exw2_ref, exb2_ref,
                exw3_ref, exb3_ref,
                mm1a_ref, mm1b_ref, mmb1_ref, mm2_ref, mmb2_ref,
                mm3_ref,
                li1f_ref, li1x_ref, lib1_ref, li2_ref, lib2_ref,
                li3_ref, lib3_ref,
                wih_ref, whh_ref, bg_ref, outw_ref, outb_ref,
                o_ref, loss_ref, scr_ref, fin_ref):
    R = _BB * _N
    erc = erc_ref[...]
    gcat = gcat_ref[...]
    gdt = gdt_ref[...]

    exb2 = exb2_ref[...]
    exb3 = exb3_ref[...]
    mmb1 = mmb1_ref[...]
    mmb2 = mmb2_ref[...]
    lib2 = lib2_ref[...]
    lib3 = lib3_ref[...]
    bg = bg_ref[...]
    outb = outb_ref[...]

    exw1i = exw1i_ref[...]
    exw1rc = exw1rc_ref[...]
    exw2 = exw2_ref[...]
    exw3 = exw3_ref[...]
    li1x = li1x_ref[...]
    lib1 = lib1_ref[...]
    mm1a = mm1a_ref[...]
    mm1b = mm1b_ref[...]
    mm2w = mm2_ref[...]
    mm3w = mm3_ref[...]
    li1f = li1f_ref[...]
    li2w = li2_ref[...]
    li3w = li3_ref[...]
    wih = wih_ref[...]
    whh = whh_ref[...]
    outw = outw_ref[...]

    eo = eo_ref[...].reshape(R, _EMBED)
    y1h = y1h_ref[...].reshape(R, _EMBED)

    x = _relu(_mx(eo, exw1i) + _mx(erc, exw1rc))
    x = _relu(_mx(x, exw2) + exb2)
    x = _mx(x, exw3) + exb3
    xli = _mx(x, li1x) + lib1

    hm = x
    h = None
    c = None
    acc = jnp.zeros((), jnp.float32)
    B = range(_BB)
    for t in range(_STEPS):
        av = _mx(hm, mm1a) + mmb1
        bv = _mx(hm, mm1b)
        for j in B:
            abj = jnp.concatenate(
                [av[j * _N:(j + 1) * _N], bv[j * _N:(j + 1) * _N]], axis=0)
            scr_ref[pl.ds(j * _E, _E), :] = _mx(gcat, abj)
        m1 = _relu(scr_ref[...])
        m2 = _relu(_mx(m1, mm2w) + mmb2)
        msgs = _mx(m2, mm3w)
        for j in B:
            fin_ref[pl.ds(j * _N, _N), :] = _mx(
                gdt, msgs[j * _E:(j + 1) * _E])
        fin = fin_ref[...]
        li1 = _relu(_mx(fin, li1f) + xli)
        li2 = _relu(_mx(li1, li2w) + lib2)
        il = _mx(li2, li3w) + lib3
        gates = _mx(il, wih) + bg
        if t > 0:
            gates = gates + _mx(h, whh)
        ig = gates[:, 0:_H]
        fg = gates[:, _H:2 * _H]
        gg = gates[:, 2 * _H:3 * _H]
        og = gates[:, 3 * _H:4 * _H]
        newc = jax.nn.sigmoid(ig) * jnp.tanh(gg)
        if t > 0:
            newc = newc + jax.nn.sigmoid(fg) * c
        c = newc
        h = jax.nn.sigmoid(og) * jnp.tanh(newc)
        hm = h
        o16 = _mx(c, outw) + outb
        mmax = jnp.max(o16, axis=1, keepdims=True)
        lse = jnp.log(jnp.sum(jnp.exp(o16 - mmax), axis=1,
                              keepdims=True)) + mmax
        acc = acc + jnp.sum((o16 - lse) * y1h)

    o_ref[...] = o16.reshape(_BB, _N, _EMBED)
    loss_ref[...] = jnp.broadcast_to(acc, (1, 1, 128)).astype(jnp.float32)


def kernel(inp, y_true, edges, row_col,
           ex_w1, ex_b1, ex_w2, ex_b2, ex_w3, ex_b3,
           mm_w1, mm_b1, mm_w2, mm_b2, mm_w3, mm_b3,
           li_w1, li_b1, li_w2, li_b2, li_w3, li_b3,
           out_w, out_b, lstm_wih, lstm_whh, lstm_bih, lstm_bhh):
    f32 = jnp.float32
    bs = inp.shape[0]
    e = edges.shape[0]
    inp = inp.astype(jnp.int32)
    y2 = y_true.astype(jnp.int32).reshape(bs, _N)

    eo = jax.nn.one_hot(inp, _EMBED, dtype=f32)
    y1h = jax.nn.one_hot(y2, _EMBED, dtype=f32)
    erc = jnp.tile(jnp.concatenate(
        [jax.nn.one_hot(row_col[:, 0], _EMBED, dtype=f32),
         jax.nn.one_hot(row_col[:, 1], _EMBED, dtype=f32)], axis=1),
        (_BB, 1))
    gsrc = jax.nn.one_hot(edges[:, 0], _N, dtype=f32)
    gdst = jax.nn.one_hot(edges[:, 1], _N, dtype=f32)
    gcat = jnp.concatenate([gsrc, gdst], axis=1).astype(_CDT)
    gdt = gdst.T.astype(_CDT)

    cd = _CDT
    exw1i = (ex_w1[:, :_EMBED].T + ex_b1[None, :]).astype(cd)
    exw1rc = ex_w1[:, _EMBED:].T.astype(cd)
    exw2 = ex_w2.T.astype(cd)
    exw3 = ex_w3.T.astype(cd)
    mm1a = mm_w1[:, :_H].T.astype(cd)
    mm1b = mm_w1[:, _H:].T.astype(cd)
    mm2 = mm_w2.T.astype(cd)
    mm3 = mm_w3.T.astype(cd)
    li1f = li_w1[:, :_H].T.astype(cd)
    li1x = li_w1[:, _H:].T.astype(cd)
    li2 = li_w2.T.astype(cd)
    li3 = li_w3.T.astype(cd)
    wih = lstm_wih.T.astype(cd)
    whh = lstm_whh.T.astype(cd)
    bg = (lstm_bih + lstm_bhh).reshape(1, 4 * _H)
    outw = jnp.zeros((_H, _EMBED), f32).at[:, :9].set(out_w.T).astype(cd)
    outb = jnp.full((1, _EMBED), _NEG, f32).at[0, :9].set(out_b)

    b2 = ex_b2.reshape(1, _H)
    b3 = ex_b3.reshape(1, _H)
    mb1 = mm_b1.reshape(1, _H)
    mb2 = mm_b2.reshape(1, _H)
    indeg = jnp.sum(gdst, axis=0)
    lb1 = jnp.tile(
        li_b1[None, :] + indeg[:, None] * (mm_b3[None, :] @ li_w1[:, :_H].T),
        (_BB, 1))
    lb2 = li_b2.reshape(1, _H)
    lb3 = li_b3.reshape(1, _H)

    full = lambda shape: pl.BlockSpec(shape, lambda i: (0,) * len(shape))
    per_b = pl.BlockSpec((_BB, _N, _EMBED), lambda i: (i, 0, 0))
    R = _BB * _N

    o_out, loss_out = pl.pallas_call(
        _rrn_kernel,
        grid=(bs // _BB,),
        in_specs=[
            per_b, per_b, full((R, 32)), full((e, 128)), full((_N, e)),
            full((_EMBED, _H)), full((32, _H)),
            full((_H, _H)), full((1, _H)), full((_H, _H)), full((1, _H)),
            full((_H, _H)), full((_H, _H)), full((1, _H)),
            full((_H, _H)), full((1, _H)), full((_H, _H)),
            full((_H, _H)), full((_H, _H)), full((R, _H)),
            full((_H, _H)), full((1, _H)), full((_H, _H)), full((1, _H)),
            full((_H, 4 * _H)), full((_H, 4 * _H)), full((1, 4 * _H)),
            full((_H, _EMBED)), full((1, _EMBED)),
        ],
        out_specs=[
            pl.BlockSpec((_BB, _N, _EMBED), lambda i: (i, 0, 0)),
            pl.BlockSpec((1, 1, 128), lambda i: (i, 0, 0)),
        ],
        out_shape=[
            jax.ShapeDtypeStruct((bs, _N, _EMBED), f32),
            jax.ShapeDtypeStruct((bs // _BB, 1, 128), f32),
        ],
        scratch_shapes=[
            pltpu.VMEM((_BB * _E, _H), f32),
            pltpu.VMEM((R, _H), f32),
        ],
        compiler_params=pltpu.CompilerParams(
            dimension_semantics=("parallel",)),
        interpret=_INTERPRET,
    )(eo, y1h, erc, gcat, gdt,
      exw1i, exw1rc, exw2, b2, exw3, b3,
      mm1a, mm1b, mb1, mm2, mb2, mm3,
      li1f, li1x, lb1, li2, lb2, li3, lb3,
      wih, whh, bg, outw, outb)

    o = o_out.reshape(bs * _N, _EMBED)[:, :9]
    l = -jnp.sum(loss_out[:, 0, 0]) / (bs * _N)
    return (o, l)


_INTERPRET = False
